# baseline (speedup 1.0000x reference)
import functools

import jax
import jax.numpy as jnp
from jax import lax
from jax.experimental import pallas as pl
from jax.experimental.pallas import tpu as pltpu
from jax.experimental.pallas import tpu_sc as plsc

H = 128
NA = 320000
NM = 10000

BLK = 16384
ROWS = BLK // 128


def _mlp_body(x_ref, w1_ref, b1_ref, w2p_ref, b2_ref, o_ref):
  x = x_ref[...]
  h = lax.dot_general(x, w1_ref[...], (((1,), (1,)), ((), ())),
                      preferred_element_type=jnp.float32)
  h = h + b1_ref[...]
  a = 0.5 * h
  h = a + a * jnp.tanh(a)
  ep = lax.dot_general(h, w2p_ref[...], (((1,), (1,)), ((), ())),
                       preferred_element_type=jnp.float32)
  e = ep[:, 0:1] + b2_ref[0]
  o_ref[...] = e.reshape(ROWS, 128)


def _mlp(s, W1, b1, W2, b2):
  grid = pl.cdiv(NA, BLK)
  return pl.pallas_call(
      _mlp_body,
      grid=(grid,),
      in_specs=[
          pl.BlockSpec((BLK, H), lambda i: (i, 0)),
          pl.BlockSpec((H, H), lambda i: (0, 0)),
          pl.BlockSpec((1, H), lambda i: (0, 0)),
          pl.BlockSpec((H, H), lambda i: (0, 0)),
          pl.BlockSpec(memory_space=pltpu.MemorySpace.SMEM),
      ],
      out_specs=pl.BlockSpec((ROWS, 128), lambda i: (i, 0)),
      out_shape=jax.ShapeDtypeStruct((NA // 128, 128), jnp.float32),
  )(s, W1, b1, W2, b2)


NS = 16
CHUNK = NA // NS
OUTP = 640
NMP = NS * OUTP


def _shuffle(x, idx):
  return jnp.take_along_axis(x, idx, axis=0, mode="promise_in_bounds")


def _segsum_body(e_hbm, b_hbm, out_hbm, ev, iv, acc, tmp, res, stash,
                 sem_e, sem_b):
  sid = lax.axis_index("s")
  base = sid * CHUNK
  cp_e = pltpu.async_copy(e_hbm.at[pl.ds(base, CHUNK)], ev, sem_e)
  cp_b = pltpu.async_copy(b_hbm.at[pl.ds(base, CHUNK)], iv, sem_b)

  zeros16 = jnp.zeros((16,), jnp.float32)

  def zbody(j, carry):
    acc[pl.ds(j * 16, 16)] = zeros16
    return carry

  lax.fori_loop(0, NMP // 16, zbody, 0)
  cp_e.wait()
  cp_b.wait()

  iota = lax.iota(jnp.int32, 16)
  nxt_idx = jnp.where(iota == 15, 15, iota + 1)
  last_lane = iota == 15
  not_last_lane = iota != 15

  UNROLL = 5

  def body(i, carry):
    for u in range(UNROLL):
      j = i * UNROLL + u
      ids = iv[pl.ds(j * 16, 16)]
      vals = ev[pl.ds(j * 16, 16)]
      c = plsc.cumsum(vals)
      nxt = _shuffle(ids, nxt_idx)
      is_last = jnp.logical_or(last_lane, ids != nxt)
      plsc.addupdate_scatter(acc, [ids], c, mask=is_last)
      plsc.addupdate_scatter(acc, [nxt], -c,
                             mask=jnp.logical_and(is_last, not_last_lane))
    return carry

  lax.fori_loop(0, CHUNK // (16 * UNROLL), body, 0)

  pltpu.sync_copy(acc, stash.at[sid])
  plsc.subcore_barrier()
  pltpu.sync_copy(stash.at[:, pl.ds(sid * OUTP, OUTP)], tmp)

  def rbody(j, carry):
    v = tmp[0, pl.ds(j * 16, 16)]
    for r in range(1, NS):
      v = v + tmp[r, pl.ds(j * 16, 16)]
    res[pl.ds(j * 16, 16)] = v
    return carry

  lax.fori_loop(0, OUTP // 16, rbody, 0)
  pltpu.sync_copy(res, out_hbm.at[pl.ds(sid * OUTP, OUTP)])


def _segsum(e_flat, batch):
  mesh = plsc.VectorSubcoreMesh(
      core_axis_name="c", subcore_axis_name="s", num_cores=1)
  f = pl.kernel(
      _segsum_body,
      out_type=jax.ShapeDtypeStruct((NMP,), jnp.float32),
      mesh=mesh,
      scratch_types=[
          pltpu.VMEM((CHUNK,), jnp.float32),
          pltpu.VMEM((CHUNK,), jnp.int32),
          pltpu.VMEM((NMP,), jnp.float32),
          pltpu.VMEM((NS, OUTP), jnp.float32),
          pltpu.VMEM((OUTP,), jnp.float32),
          pltpu.VMEM_SHARED((NS, NMP), jnp.float32),
          pltpu.SemaphoreType.DMA,
          pltpu.SemaphoreType.DMA,
      ],
      compiler_params=pltpu.CompilerParams(needs_layout_passes=False),
  )
  return f(e_flat, batch)


def kernel(s, batch, W1, b1, W2, b2):
  W2p = jnp.zeros((H, H), jnp.float32).at[0, :].set(W2[0])
  e2d = _mlp(s, W1, b1.reshape(1, H), W2p, b2)
  e_flat = e2d.reshape(NA)
  out = _segsum(e_flat, batch)
  return out[:NM].reshape(NM, 1)


# speedup vs baseline: 1.0484x; 1.0484x over previous
import functools

import jax
import jax.numpy as jnp
from jax import lax
from jax.experimental import pallas as pl
from jax.experimental.pallas import tpu as pltpu
from jax.experimental.pallas import tpu_sc as plsc

H = 128
NA = 320000
NM = 10000

BLK = 32768
ROWS = BLK // 128

NA1 = 5 * BLK
NA2 = NA - NA1

NS = 16
OUTP = 640
NMP = NS * OUTP


def _mlp_body(x_ref, w1_ref, b1_ref, w2p_ref, b2_ref, o_ref):
  x = x_ref[...]
  h = lax.dot_general(x, w1_ref[...], (((1,), (1,)), ((), ())),
                      preferred_element_type=jnp.float32)
  h = h + b1_ref[...]
  a = 0.5 * h
  h = a + a * jnp.tanh(a)
  ep = lax.dot_general(h, w2p_ref[...], (((1,), (1,)), ((), ())),
                       preferred_element_type=jnp.float32)
  e = ep[:, 0:1] + b2_ref[0]
  o_ref[...] = e.reshape(ROWS, 128)


def _mlp_part(s, W1, b1, W2p, b2, start_blk, n_atoms):
  grid = pl.cdiv(n_atoms, BLK)
  return pl.pallas_call(
      _mlp_body,
      grid=(grid,),
      in_specs=[
          pl.BlockSpec((BLK, H), lambda i, sb=start_blk: (sb + i, 0)),
          pl.BlockSpec((H, H), lambda i: (0, 0)),
          pl.BlockSpec((1, H), lambda i: (0, 0)),
          pl.BlockSpec((H, H), lambda i: (0, 0)),
          pl.BlockSpec(memory_space=pltpu.MemorySpace.SMEM),
      ],
      out_specs=pl.BlockSpec((ROWS, 128), lambda i: (i, 0)),
      out_shape=jax.ShapeDtypeStruct((n_atoms // 128, 128), jnp.float32),
  )(s, W1, b1, W2p, b2)


def _shuffle(x, idx):
  return jnp.take_along_axis(x, idx, axis=0, mode="promise_in_bounds")


def _pick_unroll(iters):
  for u in (8, 6, 5, 4, 2, 1):
    if iters % u == 0:
      return u
  return 1


def _make_segsum(cbase, n_atoms, with_prev):
  chunk = n_atoms // NS
  assert chunk % 16 == 0 and (cbase + chunk) % 8 == 0
  iters = chunk // 16
  unroll = _pick_unroll(iters)

  def body(*args):
    if with_prev:
      (e_hbm, b_hbm, prev_hbm, out_hbm, ev, iv, acc, tmp, res, pv, stash,
       sem_e, sem_b, sem_p) = args
    else:
      (e_hbm, b_hbm, out_hbm, ev, iv, acc, tmp, res, pv, stash,
       sem_e, sem_b, sem_p) = args
      prev_hbm = None

    sid = lax.axis_index("s")
    base = sid * chunk
    cp_e = pltpu.async_copy(e_hbm.at[pl.ds(base, chunk)], ev, sem_e)
    cp_b = pltpu.async_copy(b_hbm.at[pl.ds(cbase + base, chunk)], iv, sem_b)
    if with_prev:
      cp_p = pltpu.async_copy(prev_hbm.at[pl.ds(sid * OUTP, OUTP)], pv, sem_p)

    zeros16 = jnp.zeros((16,), jnp.float32)

    def zbody(j, carry):
      acc[pl.ds(j * 16, 16)] = zeros16
      return carry

    lax.fori_loop(0, NMP // 16, zbody, 0)
    cp_e.wait()
    cp_b.wait()

    iota = lax.iota(jnp.int32, 16)
    nxt_idx = jnp.where(iota == 15, 15, iota + 1)
    last_lane = iota == 15
    not_last_lane = iota != 15

    def loop_body(i, carry):
      for u in range(unroll):
        j = i * unroll + u
        ids = iv[pl.ds(j * 16, 16)]
        vals = ev[pl.ds(j * 16, 16)]
        c = plsc.cumsum(vals)
        nxt = _shuffle(ids, nxt_idx)
        is_last = jnp.logical_or(last_lane, ids != nxt)
        plsc.addupdate_scatter(acc, [ids], c, mask=is_last)
        plsc.addupdate_scatter(acc, [nxt], -c,
                               mask=jnp.logical_and(is_last, not_last_lane))
      return carry

    lax.fori_loop(0, iters // unroll, loop_body, 0)

    pltpu.sync_copy(acc, stash.at[sid])
    plsc.subcore_barrier()
    pltpu.sync_copy(stash.at[:, pl.ds(sid * OUTP, OUTP)], tmp)
    if with_prev:
      cp_p.wait()

    def rbody(j, carry):
      if with_prev:
        v = pv[pl.ds(j * 16, 16)] + tmp[0, pl.ds(j * 16, 16)]
      else:
        v = tmp[0, pl.ds(j * 16, 16)]
      for r in range(1, NS):
        v = v + tmp[r, pl.ds(j * 16, 16)]
      res[pl.ds(j * 16, 16)] = v
      return carry

    lax.fori_loop(0, OUTP // 16, rbody, 0)
    pltpu.sync_copy(res, out_hbm.at[pl.ds(sid * OUTP, OUTP)])

  mesh = plsc.VectorSubcoreMesh(
      core_axis_name="c", subcore_axis_name="s", num_cores=1)
  return pl.kernel(
      body,
      out_type=jax.ShapeDtypeStruct((NMP,), jnp.float32),
      mesh=mesh,
      scratch_types=[
          pltpu.VMEM((chunk,), jnp.float32),
          pltpu.VMEM((chunk,), jnp.int32),
          pltpu.VMEM((NMP,), jnp.float32),
          pltpu.VMEM((NS, OUTP), jnp.float32),
          pltpu.VMEM((OUTP,), jnp.float32),
          pltpu.VMEM((OUTP,), jnp.float32),
          pltpu.VMEM_SHARED((NS, NMP), jnp.float32),
          pltpu.SemaphoreType.DMA,
          pltpu.SemaphoreType.DMA,
          pltpu.SemaphoreType.DMA,
      ],
      compiler_params=pltpu.CompilerParams(needs_layout_passes=False),
  )


def kernel(s, batch, W1, b1, W2, b2):
  W2p = jnp.zeros((H, H), jnp.float32).at[0, :].set(W2[0])
  b1r = b1.reshape(1, H)
  e1 = _mlp_part(s, W1, b1r, W2p, b2, 0, NA1).reshape(NA1)
  e2 = _mlp_part(s, W1, b1r, W2p, b2, NA1 // BLK, NA2).reshape(NA2)
  p1 = _make_segsum(0, NA1, False)(e1, batch)
  out = _make_segsum(NA1, NA2, True)(e2, batch, p1)
  return out[:NM].reshape(NM, 1)


# speedup vs baseline: 1.0775x; 1.0278x over previous
import functools

import jax
import jax.numpy as jnp
from jax import lax
from jax.experimental import pallas as pl
from jax.experimental.pallas import tpu as pltpu
from jax.experimental.pallas import tpu_sc as plsc

H = 128
NA = 320000
NM = 10000

BLK = 32768
ROWS = BLK // 128

NA1 = 7 * BLK
NA2 = NA - NA1

NS = 16
OUTP = 640
NMP = NS * OUTP


def _mlp_body(x_ref, w1_ref, b1_ref, w2p_ref, b2_ref, o_ref):
  x = x_ref[...]
  h = lax.dot_general(x, w1_ref[...], (((1,), (1,)), ((), ())),
                      preferred_element_type=jnp.float32)
  h = h + b1_ref[...]
  a = 0.5 * h
  h = a + a * jnp.tanh(a)
  ep = lax.dot_general(h, w2p_ref[...], (((1,), (1,)), ((), ())),
                       preferred_element_type=jnp.float32)
  e = ep[:, 0:1] + b2_ref[0]
  o_ref[...] = e.reshape(ROWS, 128)


def _mlp_part(s, W1, b1, W2p, b2, start_blk, n_atoms):
  grid = pl.cdiv(n_atoms, BLK)
  return pl.pallas_call(
      _mlp_body,
      grid=(grid,),
      in_specs=[
          pl.BlockSpec((BLK, H), lambda i, sb=start_blk: (sb + i, 0)),
          pl.BlockSpec((H, H), lambda i: (0, 0)),
          pl.BlockSpec((1, H), lambda i: (0, 0)),
          pl.BlockSpec((H, H), lambda i: (0, 0)),
          pl.BlockSpec(memory_space=pltpu.MemorySpace.SMEM),
      ],
      out_specs=pl.BlockSpec((ROWS, 128), lambda i: (i, 0)),
      out_shape=jax.ShapeDtypeStruct((n_atoms // 128, 128), jnp.float32),
  )(s, W1, b1, W2p, b2)


def _shuffle(x, idx):
  return jnp.take_along_axis(x, idx, axis=0, mode="promise_in_bounds")


def _pick_unroll(iters):
  for u in (8, 6, 5, 4, 2, 1):
    if iters % u == 0:
      return u
  return 1


def _make_segsum(cbase, n_atoms, with_prev):
  chunk = n_atoms // NS
  assert chunk % 16 == 0 and (cbase + chunk) % 8 == 0
  iters = chunk // 16
  unroll = _pick_unroll(iters)

  def body(*args):
    if with_prev:
      (e_hbm, b_hbm, prev_hbm, out_hbm, ev, iv, acc, tmp, res, pv, stash,
       sem_e, sem_b, sem_p) = args
    else:
      (e_hbm, b_hbm, out_hbm, ev, iv, acc, tmp, res, pv, stash,
       sem_e, sem_b, sem_p) = args
      prev_hbm = None

    sid = lax.axis_index("s")
    base = sid * chunk
    cp_e = pltpu.async_copy(e_hbm.at[pl.ds(base, chunk)], ev, sem_e)
    cp_b = pltpu.async_copy(b_hbm.at[pl.ds(cbase + base, chunk)], iv, sem_b)
    if with_prev:
      cp_p = pltpu.async_copy(prev_hbm.at[pl.ds(sid * OUTP, OUTP)], pv, sem_p)

    zeros16 = jnp.zeros((16,), jnp.float32)

    def zbody(j, carry):
      acc[pl.ds(j * 16, 16)] = zeros16
      return carry

    lax.fori_loop(0, NMP // 16, zbody, 0)
    cp_e.wait()
    cp_b.wait()

    iota = lax.iota(jnp.int32, 16)
    nxt_idx = jnp.where(iota == 15, 15, iota + 1)
    last_lane = iota == 15
    not_last_lane = iota != 15

    def loop_body(i, carry):
      for u in range(unroll):
        j = i * unroll + u
        ids = iv[pl.ds(j * 16, 16)]
        vals = ev[pl.ds(j * 16, 16)]
        c = plsc.cumsum(vals)
        nxt = _shuffle(ids, nxt_idx)
        is_last = jnp.logical_or(last_lane, ids != nxt)
        plsc.addupdate_scatter(acc, [ids], c, mask=is_last)
        plsc.addupdate_scatter(acc, [nxt], -c,
                               mask=jnp.logical_and(is_last, not_last_lane))
      return carry

    lax.fori_loop(0, iters // unroll, loop_body, 0)

    pltpu.sync_copy(acc, stash.at[sid])
    plsc.subcore_barrier()
    pltpu.sync_copy(stash.at[:, pl.ds(sid * OUTP, OUTP)], tmp)
    if with_prev:
      cp_p.wait()

    def rbody(j, carry):
      if with_prev:
        v = pv[pl.ds(j * 16, 16)] + tmp[0, pl.ds(j * 16, 16)]
      else:
        v = tmp[0, pl.ds(j * 16, 16)]
      for r in range(1, NS):
        v = v + tmp[r, pl.ds(j * 16, 16)]
      res[pl.ds(j * 16, 16)] = v
      return carry

    lax.fori_loop(0, OUTP // 16, rbody, 0)
    pltpu.sync_copy(res, out_hbm.at[pl.ds(sid * OUTP, OUTP)])

  mesh = plsc.VectorSubcoreMesh(
      core_axis_name="c", subcore_axis_name="s", num_cores=1)
  return pl.kernel(
      body,
      out_type=jax.ShapeDtypeStruct((NMP,), jnp.float32),
      mesh=mesh,
      scratch_types=[
          pltpu.VMEM((chunk,), jnp.float32),
          pltpu.VMEM((chunk,), jnp.int32),
          pltpu.VMEM((NMP,), jnp.float32),
          pltpu.VMEM((NS, OUTP), jnp.float32),
          pltpu.VMEM((OUTP,), jnp.float32),
          pltpu.VMEM((OUTP,), jnp.float32),
          pltpu.VMEM_SHARED((NS, NMP), jnp.float32),
          pltpu.SemaphoreType.DMA,
          pltpu.SemaphoreType.DMA,
          pltpu.SemaphoreType.DMA,
      ],
      compiler_params=pltpu.CompilerParams(needs_layout_passes=False),
  )


def kernel(s, batch, W1, b1, W2, b2):
  W2p = jnp.zeros((H, H), jnp.float32).at[0, :].set(W2[0])
  b1r = b1.reshape(1, H)
  e1 = _mlp_part(s, W1, b1r, W2p, b2, 0, NA1).reshape(NA1)
  e2 = _mlp_part(s, W1, b1r, W2p, b2, NA1 // BLK, NA2).reshape(NA2)
  p1 = _make_segsum(0, NA1, False)(e1, batch)
  out = _make_segsum(NA1, NA2, True)(e2, batch, p1)
  return out[:NM].reshape(NM, 1)
